# baseline (speedup 1.0000x reference)
import functools

import jax
import jax.numpy as jnp
from jax import lax
from jax.experimental import pallas as pl
from jax.experimental.pallas import tpu as pltpu
from jax.experimental.pallas import tpu_sc as plsc

_N = 10000
_E = 320000
_D = 128
_NC = 2
_NS = 16
_CPT = _D // (_NC * _NS)
_CH = 6400
_NCH = _E // _CH


def _sc_segment_max(xT, src, dst):
  mesh = plsc.VectorSubcoreMesh(
      core_axis_name="c", subcore_axis_name="s",
      num_cores=_NC, num_subcores=_NS)

  scratch = (
      [pltpu.VMEM((_N,), jnp.float32) for _ in range(_CPT)]
      + [pltpu.VMEM((_N,), jnp.float32) for _ in range(_CPT)]
      + [pltpu.VMEM((_CH,), jnp.int32) for _ in range(4)]
      + [pltpu.SemaphoreType.DMA, pltpu.SemaphoreType.DMA]
  )

  @functools.partial(
      pl.kernel, mesh=mesh,
      out_type=jax.ShapeDtypeStruct((_D, _N), jnp.float32),
      scratch_types=scratch,
      compiler_params=pltpu.CompilerParams(needs_layout_passes=False),
  )
  def body(xt_hbm, src_hbm, dst_hbm, out_hbm,
           xl0, xl1, xl2, xl3, ml0, ml1, ml2, ml3,
           sb0, db0, sb1, db1, sem0, sem1):
    xls = (xl0, xl1, xl2, xl3)
    mls = (ml0, ml1, ml2, ml3)
    sbufs = (sb0, sb1)
    dbufs = (db0, db1)
    sems = (sem0, sem1)

    wid = lax.axis_index("s") * _NC + lax.axis_index("c")
    c0 = wid * _CPT

    for j in range(_CPT):
      pltpu.sync_copy(xt_hbm.at[c0 + j], xls[j])

    neg = jnp.full((16,), -jnp.inf, dtype=jnp.float32)

    def ibody(i, carry):
      for j in range(_CPT):
        mls[j][pl.ds(i * 16, 16)] = neg
      return carry

    lax.fori_loop(0, _N // 16, ibody, 0)

    def start(slot, ci):
      off = ci * _CH
      pltpu.async_copy(src_hbm.at[pl.ds(off, _CH)], sbufs[slot], sems[slot])
      pltpu.async_copy(dst_hbm.at[pl.ds(off, _CH)], dbufs[slot], sems[slot])

    def wait(slot):
      pltpu.make_async_copy(
          src_hbm.at[pl.ds(0, _CH)], sbufs[slot], sems[slot]).wait()
      pltpu.make_async_copy(
          dst_hbm.at[pl.ds(0, _CH)], dbufs[slot], sems[slot]).wait()

    def fixpoint(d, vals):
      p0 = jnp.full((16,), True)

      def cond(ps):
        return jnp.any(ps[0] | ps[1] | ps[2] | ps[3])

      def wbody(ps):
        out = []
        for j in range(_CPT):
          old = plsc.load_gather(mls[j], [d])
          new = jnp.maximum(old, vals[j])
          plsc.store_scatter(mls[j], [d], new, mask=ps[j])
          chk = plsc.load_gather(mls[j], [d])
          out.append(ps[j] & (chk < vals[j]))
        return tuple(out)

      lax.while_loop(cond, wbody, (p0, p0, p0, p0))

    _U = 16

    def process(sbuf, dbuf):
      def gbody(it, carry):
        g0 = it * _U
        ds, valss, lasts = [], [], []
        for u in range(_U):
          s = sbuf[pl.ds((g0 + u) * 16, 16)]
          d = dbuf[pl.ds((g0 + u) * 16, 16)]
          _, last = plsc.scan_count(d)
          ds.append(d)
          lasts.append(last)
          valss.append(tuple(
              plsc.load_gather(xls[j], [s]) for j in range(_CPT)))
        all_unique = functools.reduce(lambda a, b: a & b, lasts)
        dup = jnp.any(jnp.logical_not(all_unique))

        def fast():
          for u in range(_U):
            for j in range(_CPT):
              old = plsc.load_gather(mls[j], [ds[u]])
              plsc.store_scatter(mls[j], [ds[u]],
                                 jnp.maximum(old, valss[u][j]))

        def slow():
          for u in range(_U):
            fixpoint(ds[u], valss[u])

        lax.cond(dup, slow, fast)
        return carry

      lax.fori_loop(0, _CH // 16 // _U, gbody, 0)

    start(0, 0)

    def pbody(p, carry):
      base = 2 * p
      start(1, base + 1)
      wait(0)
      process(sb0, db0)

      @pl.when(base + 2 < _NCH)
      def _():
        start(0, base + 2)

      wait(1)
      process(sb1, db1)
      return carry

    lax.fori_loop(0, _NCH // 2, pbody, 0)

    for j in range(_CPT):
      pltpu.sync_copy(mls[j], out_hbm.at[c0 + j])

  return body(xT, src, dst)


def _tc_mlp_body(xt_ref, mt_ref, w1_ref, w2_ref, b_ref, o_ref):
  xt = xt_ref[...]
  agg = mt_ref[...] - xt
  agg = jnp.where(agg < -10000.0, 0.0, agg)
  dn = (((0,), (1,)), ((), ()))
  acc = lax.dot_general(xt, w1_ref[...], dn,
                        preferred_element_type=jnp.float32)
  acc = acc + lax.dot_general(agg, w2_ref[...], dn,
                              preferred_element_type=jnp.float32)
  o_ref[...] = jnp.maximum(acc + b_ref[...], 0.0)


def _tc_mlp(xT, Mt, W1, W2, b2d):
  return pl.pallas_call(
      _tc_mlp_body,
      out_shape=jax.ShapeDtypeStruct((_N, _D), jnp.float32),
  )(xT, Mt, W1, W2, b2d)


def kernel(x, edge_index, W, b):
  xT = x.T
  src = edge_index[0]
  dst = edge_index[1]
  Mt = _sc_segment_max(xT, src, dst)
  W1 = W[:, :_D]
  W2 = W[:, _D:]
  return _tc_mlp(xT, Mt, W1, W2, b.reshape(1, _D))


# speedup vs baseline: 1.5697x; 1.5697x over previous
import functools

import jax
import jax.numpy as jnp
from jax import lax
from jax.experimental import pallas as pl
from jax.experimental.pallas import tpu as pltpu
from jax.experimental.pallas import tpu_sc as plsc

_N = 10000
_E = 320000
_D = 128
_NC = 2
_NS = 16
_CPT = _D // (_NC * _NS)
_CH = 6400
_NCH = _E // _CH


def _sc_segment_max(xT, src, dst):
  mesh = plsc.VectorSubcoreMesh(
      core_axis_name="c", subcore_axis_name="s",
      num_cores=_NC, num_subcores=_NS)

  scratch = (
      [pltpu.VMEM((_N,), jnp.float32) for _ in range(_CPT)]
      + [pltpu.VMEM((_N,), jnp.float32) for _ in range(_CPT)]
      + [pltpu.VMEM((_CH,), jnp.int32) for _ in range(4)]
      + [pltpu.SemaphoreType.DMA, pltpu.SemaphoreType.DMA]
  )

  @functools.partial(
      pl.kernel, mesh=mesh,
      out_type=jax.ShapeDtypeStruct((_D, _N), jnp.float32),
      scratch_types=scratch,
      compiler_params=pltpu.CompilerParams(needs_layout_passes=False),
  )
  def body(xt_hbm, src_hbm, dst_hbm, out_hbm,
           xl0, xl1, xl2, xl3, ml0, ml1, ml2, ml3,
           sb0, db0, sb1, db1, sem0, sem1):
    xls = (xl0, xl1, xl2, xl3)
    mls = (ml0, ml1, ml2, ml3)
    sbufs = (sb0, sb1)
    dbufs = (db0, db1)
    sems = (sem0, sem1)

    wid = lax.axis_index("s") * _NC + lax.axis_index("c")
    c0 = wid * _CPT

    for j in range(_CPT):
      pltpu.sync_copy(xt_hbm.at[c0 + j], xls[j])

    neg = jnp.full((16,), -jnp.inf, dtype=jnp.float32)

    def ibody(i, carry):
      for j in range(_CPT):
        mls[j][pl.ds(i * 16, 16)] = neg
      return carry

    lax.fori_loop(0, _N // 16, ibody, 0)

    def start(slot, ci):
      off = ci * _CH
      pltpu.async_copy(src_hbm.at[pl.ds(off, _CH)], sbufs[slot], sems[slot])
      pltpu.async_copy(dst_hbm.at[pl.ds(off, _CH)], dbufs[slot], sems[slot])

    def wait(slot):
      pltpu.make_async_copy(
          src_hbm.at[pl.ds(0, _CH)], sbufs[slot], sems[slot]).wait()
      pltpu.make_async_copy(
          dst_hbm.at[pl.ds(0, _CH)], dbufs[slot], sems[slot]).wait()

    def fixpoint(d, vals):
      p0 = jnp.full((16,), True)

      def cond(ps):
        return jnp.any(ps[0] | ps[1] | ps[2] | ps[3])

      def wbody(ps):
        out = []
        for j in range(_CPT):
          old = plsc.load_gather(mls[j], [d])
          new = jnp.maximum(old, vals[j])
          plsc.store_scatter(mls[j], [d], new, mask=ps[j])
          chk = plsc.load_gather(mls[j], [d])
          out.append(ps[j] & (chk < vals[j]))
        return tuple(out)

      lax.while_loop(cond, wbody, (p0, p0, p0, p0))

    _U = 8

    def process(sbuf, dbuf):
      def gbody(it, carry):
        g0 = it * _U
        ds, valss, lasts = [], [], []
        for u in range(_U):
          s = sbuf[pl.ds((g0 + u) * 16, 16)]
          d = dbuf[pl.ds((g0 + u) * 16, 16)]
          _, last = plsc.scan_count(d)
          ds.append(d)
          lasts.append(last)
          valss.append(tuple(
              plsc.load_gather(xls[j], [s]) for j in range(_CPT)))
        all_unique = functools.reduce(lambda a, b: a & b, lasts)
        dup = jnp.any(jnp.logical_not(all_unique))

        def fast():
          for u in range(_U):
            olds = [plsc.load_gather(mls[j], [ds[u]]) for j in range(_CPT)]
            news = [jnp.maximum(olds[j], valss[u][j]) for j in range(_CPT)]
            for j in range(_CPT):
              plsc.store_scatter(mls[j], [ds[u]], news[j])

        def slow():
          for u in range(_U):
            fixpoint(ds[u], valss[u])

        lax.cond(dup, slow, fast)
        return carry

      lax.fori_loop(0, _CH // 16 // _U, gbody, 0)

    start(0, 0)

    def pbody(p, carry):
      base = 2 * p
      start(1, base + 1)
      wait(0)
      process(sb0, db0)

      @pl.when(base + 2 < _NCH)
      def _():
        start(0, base + 2)

      wait(1)
      process(sb1, db1)
      return carry

    lax.fori_loop(0, _NCH // 2, pbody, 0)

    for j in range(_CPT):
      pltpu.sync_copy(mls[j], out_hbm.at[c0 + j])

  return body(xT, src, dst)


def _tc_mlp_body(xt_ref, mt_ref, w1_ref, w2_ref, b_ref, o_ref):
  xt = xt_ref[...]
  agg = mt_ref[...] - xt
  agg = jnp.where(agg < -10000.0, 0.0, agg)
  dn = (((0,), (1,)), ((), ()))
  acc = lax.dot_general(xt, w1_ref[...], dn,
                        preferred_element_type=jnp.float32)
  acc = acc + lax.dot_general(agg, w2_ref[...], dn,
                              preferred_element_type=jnp.float32)
  o_ref[...] = jnp.maximum(acc + b_ref[...], 0.0)


def _tc_mlp(xT, Mt, W1, W2, b2d):
  return pl.pallas_call(
      _tc_mlp_body,
      out_shape=jax.ShapeDtypeStruct((_N, _D), jnp.float32),
  )(xT, Mt, W1, W2, b2d)


def kernel(x, edge_index, W, b):
  xT = x.T
  src = edge_index[0]
  dst = edge_index[1]
  Mt = _sc_segment_max(xT, src, dst)
  W1 = W[:, :_D]
  W2 = W[:, _D:]
  return _tc_mlp(xT, Mt, W1, W2, b.reshape(1, _D))


# speedup vs baseline: 1.7104x; 1.0896x over previous
import functools

import jax
import jax.numpy as jnp
from jax import lax
from jax.experimental import pallas as pl
from jax.experimental.pallas import tpu as pltpu
from jax.experimental.pallas import tpu_sc as plsc

_N = 10000
_E = 320000
_D = 128
_NC = 2
_NS = 16
_CPT = _D // (_NC * _NS)
_CH = 1280
_NCH = _E // _CH


def _sc_segment_max(xT, src, dst):
  mesh = plsc.VectorSubcoreMesh(
      core_axis_name="c", subcore_axis_name="s",
      num_cores=_NC, num_subcores=_NS)

  scratch = (
      [pltpu.VMEM((_N,), jnp.float32) for _ in range(_CPT)]
      + [pltpu.VMEM((_N,), jnp.float32) for _ in range(2 * _CPT)]
      + [pltpu.VMEM((_CH,), jnp.int32) for _ in range(4)]
      + [pltpu.SemaphoreType.DMA, pltpu.SemaphoreType.DMA]
  )

  @functools.partial(
      pl.kernel, mesh=mesh,
      out_type=jax.ShapeDtypeStruct((_D, _N), jnp.float32),
      scratch_types=scratch,
      compiler_params=pltpu.CompilerParams(needs_layout_passes=False),
  )
  def body(xt_hbm, src_hbm, dst_hbm, out_hbm,
           xl0, xl1, xl2, xl3, ma0, ma1, ma2, ma3, mb0, mb1, mb2, mb3,
           sb0, db0, sb1, db1, sem0, sem1):
    xls = (xl0, xl1, xl2, xl3)
    banks = ((ma0, ma1, ma2, ma3), (mb0, mb1, mb2, mb3))
    sbufs = (sb0, sb1)
    dbufs = (db0, db1)
    sems = (sem0, sem1)

    wid = lax.axis_index("s") * _NC + lax.axis_index("c")
    c0 = wid * _CPT

    for j in range(_CPT):
      pltpu.sync_copy(xt_hbm.at[c0 + j], xls[j])

    neg = jnp.full((16,), -jnp.inf, dtype=jnp.float32)

    def ibody(i, carry):
      for mls in banks:
        for j in range(_CPT):
          mls[j][pl.ds(i * 16, 16)] = neg
      return carry

    lax.fori_loop(0, _N // 16, ibody, 0)

    def start(slot, ci):
      off = ci * _CH
      pltpu.async_copy(src_hbm.at[pl.ds(off, _CH)], sbufs[slot], sems[slot])
      pltpu.async_copy(dst_hbm.at[pl.ds(off, _CH)], dbufs[slot], sems[slot])

    def wait(slot):
      pltpu.make_async_copy(
          src_hbm.at[pl.ds(0, _CH)], sbufs[slot], sems[slot]).wait()
      pltpu.make_async_copy(
          dst_hbm.at[pl.ds(0, _CH)], dbufs[slot], sems[slot]).wait()

    def fixpoint(mls, d, vals):
      p0 = jnp.full((16,), True)

      def cond(ps):
        return jnp.any(ps[0] | ps[1] | ps[2] | ps[3])

      def wbody(ps):
        out = []
        for j in range(_CPT):
          old = plsc.load_gather(mls[j], [d])
          new = jnp.maximum(old, vals[j])
          plsc.store_scatter(mls[j], [d], new, mask=ps[j])
          chk = plsc.load_gather(mls[j], [d])
          out.append(ps[j] & (chk < vals[j]))
        return tuple(out)

      lax.while_loop(cond, wbody, (p0, p0, p0, p0))

    _U = 8

    def process(sbuf, dbuf):
      def gbody(it, carry):
        g0 = it * _U
        ds, valss, lasts = [], [], []
        for u in range(_U):
          s = sbuf[pl.ds((g0 + u) * 16, 16)]
          d = dbuf[pl.ds((g0 + u) * 16, 16)]
          _, last = plsc.scan_count(d)
          ds.append(d)
          lasts.append(last)
          valss.append(tuple(
              plsc.load_gather(xls[j], [s]) for j in range(_CPT)))
        all_unique = functools.reduce(lambda a, b: a & b, lasts)
        dup = jnp.any(jnp.logical_not(all_unique))

        def fast():
          for u in range(0, _U, 2):
            olds = [plsc.load_gather(banks[0][j], [ds[u]])
                    for j in range(_CPT)]
            olds += [plsc.load_gather(banks[1][j], [ds[u + 1]])
                     for j in range(_CPT)]
            news = [jnp.maximum(olds[j], valss[u][j]) for j in range(_CPT)]
            news += [jnp.maximum(olds[_CPT + j], valss[u + 1][j])
                     for j in range(_CPT)]
            for j in range(_CPT):
              plsc.store_scatter(banks[0][j], [ds[u]], news[j])
              plsc.store_scatter(banks[1][j], [ds[u + 1]], news[_CPT + j])

        def slow():
          for u in range(_U):
            fixpoint(banks[u % 2], ds[u], valss[u])

        lax.cond(dup, slow, fast)
        return carry

      lax.fori_loop(0, _CH // 16 // _U, gbody, 0)

    start(0, 0)

    def pbody(p, carry):
      base = 2 * p
      start(1, base + 1)
      wait(0)
      process(sb0, db0)

      @pl.when(base + 2 < _NCH)
      def _():
        start(0, base + 2)

      wait(1)
      process(sb1, db1)
      return carry

    lax.fori_loop(0, _NCH // 2, pbody, 0)

    def mbody(i, carry):
      sl = pl.ds(i * 16, 16)
      for j in range(_CPT):
        banks[0][j][sl] = jnp.maximum(banks[0][j][sl], banks[1][j][sl])
      return carry

    lax.fori_loop(0, _N // 16, mbody, 0)
    for j in range(_CPT):
      pltpu.sync_copy(banks[0][j], out_hbm.at[c0 + j])

  return body(xT, src, dst)


def _tc_mlp_body(xt_ref, mt_ref, w1_ref, w2_ref, b_ref, o_ref):
  xt = xt_ref[...]
  agg = mt_ref[...] - xt
  agg = jnp.where(agg < -10000.0, 0.0, agg)
  dn = (((0,), (1,)), ((), ()))
  acc = lax.dot_general(xt, w1_ref[...], dn,
                        preferred_element_type=jnp.float32)
  acc = acc + lax.dot_general(agg, w2_ref[...], dn,
                              preferred_element_type=jnp.float32)
  o_ref[...] = jnp.maximum(acc + b_ref[...], 0.0)


def _tc_mlp(xT, Mt, W1, W2, b2d):
  return pl.pallas_call(
      _tc_mlp_body,
      out_shape=jax.ShapeDtypeStruct((_N, _D), jnp.float32),
  )(xT, Mt, W1, W2, b2d)


def kernel(x, edge_index, W, b):
  xT = x.T
  src = edge_index[0]
  dst = edge_index[1]
  Mt = _sc_segment_max(xT, src, dst)
  W1 = W[:, :_D]
  W2 = W[:, _D:]
  return _tc_mlp(xT, Mt, W1, W2, b.reshape(1, _D))


# speedup vs baseline: 1.9909x; 1.1640x over previous
import functools

import jax
import jax.numpy as jnp
from jax import lax
from jax.experimental import pallas as pl
from jax.experimental.pallas import tpu as pltpu
from jax.experimental.pallas import tpu_sc as plsc

_N = 10000
_E = 320000
_D = 128
_NC = 2
_NS = 16
_CPT = _D // (_NC * _NS)
_CH = 1280
_NCH = _E // _CH


def _sc_segment_max(xT, edge_index):
  mesh = plsc.VectorSubcoreMesh(
      core_axis_name="c", subcore_axis_name="s",
      num_cores=_NC, num_subcores=_NS)

  scratch = (
      [pltpu.VMEM((_N,), jnp.float32) for _ in range(_CPT)]
      + [pltpu.VMEM((_N,), jnp.float32) for _ in range(2 * _CPT)]
      + [pltpu.VMEM((2, _CH), jnp.int32) for _ in range(2)]
      + [pltpu.SemaphoreType.DMA, pltpu.SemaphoreType.DMA]
  )

  @functools.partial(
      pl.kernel, mesh=mesh,
      out_type=jax.ShapeDtypeStruct((_D, _N), jnp.float32),
      scratch_types=scratch,
      compiler_params=pltpu.CompilerParams(needs_layout_passes=False),
  )
  def body(xt_hbm, ei_hbm, out_hbm,
           xl0, xl1, xl2, xl3, ma0, ma1, ma2, ma3, mb0, mb1, mb2, mb3,
           eb0, eb1, sem0, sem1):
    xls = (xl0, xl1, xl2, xl3)
    banks = ((ma0, ma1, ma2, ma3), (mb0, mb1, mb2, mb3))
    ebufs = (eb0, eb1)
    sems = (sem0, sem1)

    wid = lax.axis_index("s") * _NC + lax.axis_index("c")
    c0 = wid * _CPT

    for j in range(_CPT):
      pltpu.sync_copy(xt_hbm.at[c0 + j], xls[j])

    neg = jnp.full((16,), -jnp.inf, dtype=jnp.float32)

    def ibody(i, carry):
      for mls in banks:
        for j in range(_CPT):
          mls[j][pl.ds(i * 16, 16)] = neg
      return carry

    lax.fori_loop(0, _N // 16, ibody, 0)

    def start(slot, ci):
      off = ci * _CH
      pltpu.async_copy(
          ei_hbm.at[:, pl.ds(off, _CH)], ebufs[slot], sems[slot])

    def wait(slot):
      pltpu.make_async_copy(
          ei_hbm.at[:, pl.ds(0, _CH)], ebufs[slot], sems[slot]).wait()

    def fixpoint2(dA, valsA, dB, valsB):
      p0 = jnp.full((16,), True)

      def cond(ps):
        return jnp.any(functools.reduce(lambda a, b: a | b, ps))

      def wbody(ps):
        olds = [plsc.load_gather(banks[0][j], [dA]) for j in range(_CPT)]
        olds += [plsc.load_gather(banks[1][j], [dB]) for j in range(_CPT)]
        news = [jnp.maximum(olds[j], valsA[j]) for j in range(_CPT)]
        news += [jnp.maximum(olds[_CPT + j], valsB[j]) for j in range(_CPT)]
        for j in range(_CPT):
          plsc.store_scatter(banks[0][j], [dA], news[j], mask=ps[j])
          plsc.store_scatter(banks[1][j], [dB], news[_CPT + j],
                             mask=ps[_CPT + j])
        chks = [plsc.load_gather(banks[0][j], [dA]) for j in range(_CPT)]
        chks += [plsc.load_gather(banks[1][j], [dB]) for j in range(_CPT)]
        out = [ps[j] & (chks[j] < valsA[j]) for j in range(_CPT)]
        out += [ps[_CPT + j] & (chks[_CPT + j] < valsB[j])
                for j in range(_CPT)]
        return tuple(out)

      lax.while_loop(cond, wbody, (p0,) * (2 * _CPT))

    _U = 8

    def process(ebuf):
      def gbody(it, carry):
        g0 = it * _U
        ds, valss, lasts = [], [], []
        for u in range(_U):
          s = ebuf[0, pl.ds((g0 + u) * 16, 16)]
          d = ebuf[1, pl.ds((g0 + u) * 16, 16)]
          _, last = plsc.scan_count(d)
          ds.append(d)
          lasts.append(last)
          valss.append(tuple(
              plsc.load_gather(xls[j], [s]) for j in range(_CPT)))
        all_unique = functools.reduce(lambda a, b: a & b, lasts)
        dup = jnp.any(jnp.logical_not(all_unique))

        def fast():
          for u in range(0, _U, 2):
            olds = [plsc.load_gather(banks[0][j], [ds[u]])
                    for j in range(_CPT)]
            olds += [plsc.load_gather(banks[1][j], [ds[u + 1]])
                     for j in range(_CPT)]
            news = [jnp.maximum(olds[j], valss[u][j]) for j in range(_CPT)]
            news += [jnp.maximum(olds[_CPT + j], valss[u + 1][j])
                     for j in range(_CPT)]
            for j in range(_CPT):
              plsc.store_scatter(banks[0][j], [ds[u]], news[j])
              plsc.store_scatter(banks[1][j], [ds[u + 1]], news[_CPT + j])

        def slow():
          for u in range(0, _U, 2):
            fixpoint2(ds[u], valss[u], ds[u + 1], valss[u + 1])

        lax.cond(dup, slow, fast)
        return carry

      lax.fori_loop(0, _CH // 16 // _U, gbody, 0)

    start(0, 0)

    def pbody(p, carry):
      base = 2 * p
      start(1, base + 1)
      wait(0)
      process(eb0)

      @pl.when(base + 2 < _NCH)
      def _():
        start(0, base + 2)

      wait(1)
      process(eb1)
      return carry

    lax.fori_loop(0, _NCH // 2, pbody, 0)

    def mbody(i, carry):
      sl = pl.ds(i * 16, 16)
      for j in range(_CPT):
        banks[0][j][sl] = jnp.maximum(banks[0][j][sl], banks[1][j][sl])
      return carry

    lax.fori_loop(0, _N // 16, mbody, 0)
    for j in range(_CPT):
      pltpu.sync_copy(banks[0][j], out_hbm.at[c0 + j])

  return body(xT, edge_index)


def _tc_mlp_body(xt_ref, mt_ref, w1_ref, w2_ref, b_ref, o_ref):
  xt = xt_ref[...]
  agg = mt_ref[...] - xt
  agg = jnp.where(agg < -10000.0, 0.0, agg)
  dn = (((0,), (1,)), ((), ()))
  acc = lax.dot_general(xt, w1_ref[...], dn,
                        preferred_element_type=jnp.float32)
  acc = acc + lax.dot_general(agg, w2_ref[...], dn,
                              preferred_element_type=jnp.float32)
  o_ref[...] = jnp.maximum(acc + b_ref[...], 0.0)


def _tc_mlp(xT, Mt, W1, W2, b2d):
  return pl.pallas_call(
      _tc_mlp_body,
      out_shape=jax.ShapeDtypeStruct((_N, _D), jnp.float32),
  )(xT, Mt, W1, W2, b2d)


def kernel(x, edge_index, W, b):
  xT = x.T
  Mt = _sc_segment_max(xT, edge_index)
  W1 = W[:, :_D]
  W2 = W[:, _D:]
  return _tc_mlp(xT, Mt, W1, W2, b.reshape(1, _D))
